# baseline (speedup 1.0000x reference)
import functools

import jax
import jax.numpy as jnp
from jax import lax
from jax.experimental import pallas as pl
from jax.experimental.pallas import tpu as pltpu
from jax.experimental.pallas import tpu_sc as plsc

B = 32
CTX = 96
TD = 321
ED = 32
FEAT = TD * ED
OUT_D = TD + FEAT

NC = 2
NS = 16


def _mesh():
    return plsc.VectorSubcoreMesh(
        core_axis_name="c", subcore_axis_name="s",
        num_cores=NC, num_subcores=NS)


def _wid():
    return lax.axis_index("s") * NC + lax.axis_index("c")


def _gather_kernel(tdi_hbm, table_hbm, emb_hbm, idx_v, rows_v, sem):
    wid = _wid()
    pltpu.sync_copy(tdi_hbm.at[wid], idx_v)
    pltpu.async_copy(table_hbm.at[idx_v], rows_v, sem).wait()
    pltpu.sync_copy(rows_v, emb_hbm.at[wid])


CTX_BLK = 96
B_BLK = 2


def _expand_kernel(past_ref, emb_ref, out_ref):
    for i in range(B_BLK):
        past = past_ref[i]
        feat = emb_ref[i]
        feat_b = jnp.broadcast_to(feat, (CTX_BLK, FEAT))
        out_ref[i] = jnp.concatenate([past, feat_b], axis=-1)


def kernel(past_target_cdf, target_dimension_indicator, feat_idx_emb_weight):
    tdi = target_dimension_indicator.astype(jnp.int32)

    gather = functools.partial(
        pl.kernel,
        out_type=jax.ShapeDtypeStruct((B, TD, ED), jnp.float32),
        mesh=_mesh(),
        scratch_types=[
            pltpu.VMEM((TD,), jnp.int32),
            pltpu.VMEM((TD, ED), jnp.float32),
            pltpu.SemaphoreType.DMA,
        ],
        compiler_params=pltpu.CompilerParams(use_tc_tiling_on_sc=False),
    )(_gather_kernel)
    emb = gather(tdi, feat_idx_emb_weight)

    emb2 = emb.reshape(B, 1, FEAT)

    return pl.pallas_call(
        _expand_kernel,
        grid=(B // B_BLK,),
        in_specs=[
            pl.BlockSpec((B_BLK, CTX_BLK, TD), lambda b: (b, 0, 0)),
            pl.BlockSpec((B_BLK, 1, FEAT), lambda b: (b, 0, 0)),
        ],
        out_specs=pl.BlockSpec((B_BLK, CTX_BLK, OUT_D), lambda b: (b, 0, 0)),
        out_shape=jax.ShapeDtypeStruct((B, CTX, OUT_D), jnp.float32),
    )(past_target_cdf, emb2)


# speedup vs baseline: 1.0023x; 1.0023x over previous
import functools

import jax
import jax.numpy as jnp
from jax import lax
from jax.experimental import pallas as pl
from jax.experimental.pallas import tpu as pltpu
from jax.experimental.pallas import tpu_sc as plsc

B = 32
CTX = 96
TD = 321
ED = 32
FEAT = TD * ED
OUT_D = TD + FEAT

NC = 2
NS = 16


def _mesh():
    return plsc.VectorSubcoreMesh(
        core_axis_name="c", subcore_axis_name="s",
        num_cores=NC, num_subcores=NS)


def _wid():
    return lax.axis_index("s") * NC + lax.axis_index("c")


def _gather_kernel(tdi_hbm, table_hbm, emb_hbm, idx_v, rows_v, sem):
    wid = _wid()
    pltpu.sync_copy(tdi_hbm.at[wid], idx_v)
    pltpu.async_copy(table_hbm.at[idx_v], rows_v, sem).wait()
    pltpu.sync_copy(rows_v, emb_hbm.at[wid])


CTX_BLK = 96
B_BLK = 2


def _expand_kernel(past_ref, emb_ref, out_ref):
    for i in range(B_BLK):
        feat = emb_ref[i]
        out_ref[i, :, :TD] = past_ref[i]
        out_ref[i, :, TD:] = jnp.broadcast_to(feat, (CTX_BLK, FEAT))


def kernel(past_target_cdf, target_dimension_indicator, feat_idx_emb_weight):
    tdi = target_dimension_indicator.astype(jnp.int32)

    gather = functools.partial(
        pl.kernel,
        out_type=jax.ShapeDtypeStruct((B, TD, ED), jnp.float32),
        mesh=_mesh(),
        scratch_types=[
            pltpu.VMEM((TD,), jnp.int32),
            pltpu.VMEM((TD, ED), jnp.float32),
            pltpu.SemaphoreType.DMA,
        ],
        compiler_params=pltpu.CompilerParams(use_tc_tiling_on_sc=False),
    )(_gather_kernel)
    emb = gather(tdi, feat_idx_emb_weight)

    emb2 = emb.reshape(B, 1, FEAT)

    return pl.pallas_call(
        _expand_kernel,
        grid=(B // B_BLK,),
        in_specs=[
            pl.BlockSpec((B_BLK, CTX_BLK, TD), lambda b: (b, 0, 0)),
            pl.BlockSpec((B_BLK, 1, FEAT), lambda b: (b, 0, 0)),
        ],
        out_specs=pl.BlockSpec((B_BLK, CTX_BLK, OUT_D), lambda b: (b, 0, 0)),
        out_shape=jax.ShapeDtypeStruct((B, CTX, OUT_D), jnp.float32),
    )(past_target_cdf, emb2)
